# Initial kernel scaffold; baseline (speedup 1.0000x reference)
import jax
import jax.numpy as jnp
from jax.experimental import pallas as pl


def kernel(p, x, o, params):
    raise NotImplementedError("write your pallas kernel here")


# speedup vs baseline: 9.1673x; 9.1673x over previous
import functools

import jax
import jax.numpy as jnp
from jax import lax
from jax.experimental import pallas as pl
from jax.experimental.pallas import tpu as pltpu
from jax.experimental.pallas import tpu_sc as plsc

N = 8192
C = 256
NS = 16
SEG = 2048
NSEG = 4
HID = 32
SHARE = 8
EPS = 1e-5
EDGES = N * NS

_QKV_BLK = 256
_KNN_QB = 256
_QB = 128


def _qkv_body(x_ref, wq_ref, bq_ref, wk_ref, bk_ref, wv_ref, bv_ref,
              xq_ref, xk_ref, xv_ref):
    x = x_ref[...]
    xq_ref[...] = jnp.dot(x, wq_ref[...], preferred_element_type=jnp.float32) + bq_ref[...]
    xk_ref[...] = jnp.dot(x, wk_ref[...], preferred_element_type=jnp.float32) + bk_ref[...]
    xv_ref[...] = jnp.dot(x, wv_ref[...], preferred_element_type=jnp.float32) + bv_ref[...]


def _qkv(x, WqT, bq, WkT, bk, WvT, bv):
    grid = N // _QKV_BLK
    blk = pl.BlockSpec((_QKV_BLK, C), lambda i: (i, 0))
    wspec = pl.BlockSpec((C, C), lambda i: (0, 0))
    bspec = pl.BlockSpec((1, C), lambda i: (0, 0))
    return pl.pallas_call(
        _qkv_body,
        grid=(grid,),
        in_specs=[blk, wspec, bspec, wspec, bspec, wspec, bspec],
        out_specs=[blk, blk, blk],
        out_shape=[jax.ShapeDtypeStruct((N, C), jnp.float32)] * 3,
    )(x, WqT, bq, WkT, bk, WvT, bv)


def _knn_body(p_ref, pT_ref, idx_ref):
    s = pl.program_id(0)
    q = p_ref[...]
    sT = pT_ref[...]
    sq_q = q[:, 0:1] * q[:, 0:1] + q[:, 1:2] * q[:, 1:2] + q[:, 2:3] * q[:, 2:3]
    sq_s = (sT[0:1, :] * sT[0:1, :] + sT[1:2, :] * sT[1:2, :]
            + sT[2:3, :] * sT[2:3, :])
    dot = jnp.dot(q.astype(jnp.bfloat16), sT.astype(jnp.bfloat16),
                  preferred_element_type=jnp.float32)
    d2 = sq_q + sq_s - 2.0 * dot
    loc = lax.broadcasted_iota(jnp.int32, (_KNN_QB, SEG), 1)
    big = jnp.int32(2 ** 30)
    cols = []
    for _ in range(NS):
        m = jnp.min(d2, axis=1, keepdims=True)
        idxc = jnp.min(jnp.where(d2 == m, loc, big), axis=1, keepdims=True)
        cols.append(idxc + s * SEG)
        d2 = jnp.where(loc == idxc, jnp.inf, d2)
    idx_ref[...] = jnp.concatenate(cols, axis=1)


def _knn(p, pT):
    qpb = SEG // _KNN_QB
    return pl.pallas_call(
        _knn_body,
        grid=(NSEG, qpb),
        in_specs=[
            pl.BlockSpec((_KNN_QB, 3), lambda s, j: (s * qpb + j, 0)),
            pl.BlockSpec((3, SEG), lambda s, j: (0, s)),
        ],
        out_specs=pl.BlockSpec((_KNN_QB, NS), lambda s, j: (s * qpb + j, 0)),
        out_shape=jax.ShapeDtypeStruct((N, NS), jnp.int32),
    )(p, pT)


_GCHUNK = 128
_NW = 32


def _sc_gather(xk, xv, p16, idxf):
    per_w = EDGES // _NW
    nchunk = per_w // _GCHUNK
    mesh = plsc.VectorSubcoreMesh(core_axis_name="c", subcore_axis_name="s")

    @functools.partial(
        pl.kernel,
        mesh=mesh,
        out_type=[
            jax.ShapeDtypeStruct((EDGES, C), jnp.float32),
            jax.ShapeDtypeStruct((EDGES, C), jnp.float32),
            jax.ShapeDtypeStruct((EDGES, 128), jnp.float32),
        ],
        scratch_types=[
            pltpu.VMEM((_GCHUNK,), jnp.int32),
            pltpu.VMEM((_GCHUNK, C), jnp.float32),
            pltpu.VMEM((_GCHUNK, C), jnp.float32),
            pltpu.VMEM((_GCHUNK, 128), jnp.float32),
            pltpu.SemaphoreType.DMA,
        ],
    )
    def gk(xk_hbm, xv_hbm, p16_hbm, idx_hbm, xkg_hbm, xvg_hbm, pg_hbm,
           idx_v, bufk, bufv, bufp, sem):
        wid = lax.axis_index("s") * 2 + lax.axis_index("c")

        def body(i, carry):
            base = wid * per_w + i * _GCHUNK
            pltpu.sync_copy(idx_hbm.at[pl.ds(base, _GCHUNK)], idx_v)
            pltpu.async_copy(xk_hbm.at[idx_v], bufk, sem).wait()
            pltpu.sync_copy(bufk, xkg_hbm.at[pl.ds(base, _GCHUNK)])
            pltpu.async_copy(xv_hbm.at[idx_v], bufv, sem).wait()
            pltpu.sync_copy(bufv, xvg_hbm.at[pl.ds(base, _GCHUNK)])
            pltpu.async_copy(p16_hbm.at[idx_v], bufp, sem).wait()
            pltpu.sync_copy(bufp, pg_hbm.at[pl.ds(base, _GCHUNK)])
            return carry

        lax.fori_loop(0, nchunk, body, 0)

    return gk(xk, xv, p16, idxf)


def _hstats_body(pg_ref, p_ref, wp1t_ref, bp1_ref, st_ref):
    @pl.when(pl.program_id(0) == 0)
    def _():
        st_ref[...] = jnp.zeros_like(st_ref)

    qb = p_ref.shape[0]
    p_r = pg_ref[:, :, :3] - jnp.reshape(p_ref[...], (qb, 1, 3))
    h = jnp.dot(p_r.reshape(qb * NS, 3), wp1t_ref[...],
                preferred_element_type=jnp.float32) + bp1_ref[...]
    st_ref[0:1, 0:3] += jnp.sum(h, axis=0, keepdims=True)
    st_ref[1:2, 0:3] += jnp.sum(h * h, axis=0, keepdims=True)


def _hstats(pg3, p, Wp1T, bp1):
    grid = N // 256
    return pl.pallas_call(
        _hstats_body,
        grid=(grid,),
        in_specs=[
            pl.BlockSpec((256, NS, 16), lambda i: (i, 0, 0)),
            pl.BlockSpec((256, 3), lambda i: (i, 0)),
            pl.BlockSpec((3, 3), lambda i: (0, 0)),
            pl.BlockSpec((1, 3), lambda i: (0, 0)),
        ],
        out_specs=pl.BlockSpec((8, 128), lambda i: (0, 0)),
        out_shape=jax.ShapeDtypeStruct((8, 128), jnp.float32),
    )(pg3, p, Wp1T, bp1)


def _pe_block(pg_b, p_b, wp1t, bp1, s1, t1, wp2t, bp2):
    qb = p_b.shape[0]
    p_r = pg_b[:, :, :3] - jnp.reshape(p_b, (qb, 1, 3))
    h = jnp.dot(p_r.reshape(qb * NS, 3), wp1t,
                preferred_element_type=jnp.float32) + bp1
    h = jnp.maximum(h * s1 + t1, 0.0)
    pe = jnp.dot(h, wp2t, preferred_element_type=jnp.float32) + bp2
    return pe.reshape(qb, NS, C)


def _rstats_body(pg_ref, p_ref, xq_ref, xkg_ref, wp1t_ref, bp1_ref,
                 s1_ref, t1_ref, wp2t_ref, bp2_ref, st_ref):
    @pl.when(pl.program_id(0) == 0)
    def _():
        st_ref[...] = jnp.zeros_like(st_ref)

    pe = _pe_block(pg_ref[...], p_ref[...], wp1t_ref[...], bp1_ref[...],
                   s1_ref[...], t1_ref[...], wp2t_ref[...], bp2_ref[...])
    r = xkg_ref[...] - jnp.reshape(xq_ref[...], (_QB, 1, C)) + pe
    r2 = r.reshape(_QB * NS, C)
    st_ref[0:1, :] += jnp.sum(r2, axis=0, keepdims=True)
    st_ref[1:2, :] += jnp.sum(r2 * r2, axis=0, keepdims=True)


def _rstats(pg3, p, xq, xkg3, Wp1T, bp1, s1, t1, Wp2T, bp2):
    grid = N // _QB
    return pl.pallas_call(
        _rstats_body,
        grid=(grid,),
        in_specs=[
            pl.BlockSpec((_QB, NS, 16), lambda i: (i, 0, 0)),
            pl.BlockSpec((_QB, 3), lambda i: (i, 0)),
            pl.BlockSpec((_QB, C), lambda i: (i, 0)),
            pl.BlockSpec((_QB, NS, C), lambda i: (i, 0, 0)),
            pl.BlockSpec((3, 3), lambda i: (0, 0)),
            pl.BlockSpec((1, 3), lambda i: (0, 0)),
            pl.BlockSpec((1, 3), lambda i: (0, 0)),
            pl.BlockSpec((1, 3), lambda i: (0, 0)),
            pl.BlockSpec((3, C), lambda i: (0, 0)),
            pl.BlockSpec((1, C), lambda i: (0, 0)),
        ],
        out_specs=pl.BlockSpec((8, C), lambda i: (0, 0)),
        out_shape=jax.ShapeDtypeStruct((8, C), jnp.float32),
    )(pg3, p, xq, xkg3, Wp1T, bp1, s1, t1, Wp2T, bp2)


def _w1_body(pg_ref, p_ref, xq_ref, xkg_ref, wp1t_ref, bp1_ref,
             s1_ref, t1_ref, wp2t_ref, bp2_ref, s2_ref, t2_ref,
             ww1t_ref, bw1_ref, w1_ref, st_ref):
    @pl.when(pl.program_id(0) == 0)
    def _():
        st_ref[...] = jnp.zeros_like(st_ref)

    pe = _pe_block(pg_ref[...], p_ref[...], wp1t_ref[...], bp1_ref[...],
                   s1_ref[...], t1_ref[...], wp2t_ref[...], bp2_ref[...])
    r = xkg_ref[...] - jnp.reshape(xq_ref[...], (_QB, 1, C)) + pe
    r2 = r.reshape(_QB * NS, C)
    w = jnp.maximum(r2 * s2_ref[...] + t2_ref[...], 0.0)
    w1 = jnp.dot(w, ww1t_ref[...], preferred_element_type=jnp.float32) + bw1_ref[...]
    w1_ref[...] = w1.reshape(_QB, NS, HID)
    st_ref[0:1, 0:HID] += jnp.sum(w1, axis=0, keepdims=True)
    st_ref[1:2, 0:HID] += jnp.sum(w1 * w1, axis=0, keepdims=True)


def _w1pass(pg3, p, xq, xkg3, Wp1T, bp1, s1, t1, Wp2T, bp2, s2, t2, Ww1T, bw1):
    grid = N // _QB
    return pl.pallas_call(
        _w1_body,
        grid=(grid,),
        in_specs=[
            pl.BlockSpec((_QB, NS, 16), lambda i: (i, 0, 0)),
            pl.BlockSpec((_QB, 3), lambda i: (i, 0)),
            pl.BlockSpec((_QB, C), lambda i: (i, 0)),
            pl.BlockSpec((_QB, NS, C), lambda i: (i, 0, 0)),
            pl.BlockSpec((3, 3), lambda i: (0, 0)),
            pl.BlockSpec((1, 3), lambda i: (0, 0)),
            pl.BlockSpec((1, 3), lambda i: (0, 0)),
            pl.BlockSpec((1, 3), lambda i: (0, 0)),
            pl.BlockSpec((3, C), lambda i: (0, 0)),
            pl.BlockSpec((1, C), lambda i: (0, 0)),
            pl.BlockSpec((1, C), lambda i: (0, 0)),
            pl.BlockSpec((1, C), lambda i: (0, 0)),
            pl.BlockSpec((C, HID), lambda i: (0, 0)),
            pl.BlockSpec((1, HID), lambda i: (0, 0)),
        ],
        out_specs=[
            pl.BlockSpec((_QB, NS, HID), lambda i: (i, 0, 0)),
            pl.BlockSpec((8, 128), lambda i: (0, 0)),
        ],
        out_shape=[
            jax.ShapeDtypeStruct((N, NS, HID), jnp.float32),
            jax.ShapeDtypeStruct((8, 128), jnp.float32),
        ],
    )(pg3, p, xq, xkg3, Wp1T, bp1, s1, t1, Wp2T, bp2, s2, t2, Ww1T, bw1)


def _final_body(pg_ref, p_ref, w1_ref, xvg_ref, wp1t_ref, bp1_ref,
                s1_ref, t1_ref, wp2t_ref, bp2_ref, s3_ref, t3_ref,
                ww2t_ref, bw2_ref, out_ref):
    y = jnp.maximum(w1_ref[...].reshape(_QB * NS, HID) * s3_ref[...] + t3_ref[...], 0.0)
    w2 = jnp.dot(y, ww2t_ref[...], preferred_element_type=jnp.float32) + bw2_ref[...]
    w2 = w2.reshape(_QB, NS, HID)
    m = jnp.max(w2, axis=1, keepdims=True)
    e = jnp.exp(w2 - m)
    sm = e / jnp.sum(e, axis=1, keepdims=True)
    pe = _pe_block(pg_ref[...], p_ref[...], wp1t_ref[...], bp1_ref[...],
                   s1_ref[...], t1_ref[...], wp2t_ref[...], bp2_ref[...])
    xvp = xvg_ref[...] + pe
    wf = jnp.concatenate([sm] * SHARE, axis=2)
    out_ref[...] = jnp.sum(xvp * wf, axis=1)


def _final(pg3, p, w1, xvg3, Wp1T, bp1, s1, t1, Wp2T, bp2, s3, t3, Ww2T, bw2):
    grid = N // _QB
    return pl.pallas_call(
        _final_body,
        grid=(grid,),
        in_specs=[
            pl.BlockSpec((_QB, NS, 16), lambda i: (i, 0, 0)),
            pl.BlockSpec((_QB, 3), lambda i: (i, 0)),
            pl.BlockSpec((_QB, NS, HID), lambda i: (i, 0, 0)),
            pl.BlockSpec((_QB, NS, C), lambda i: (i, 0, 0)),
            pl.BlockSpec((3, 3), lambda i: (0, 0)),
            pl.BlockSpec((1, 3), lambda i: (0, 0)),
            pl.BlockSpec((1, 3), lambda i: (0, 0)),
            pl.BlockSpec((1, 3), lambda i: (0, 0)),
            pl.BlockSpec((3, C), lambda i: (0, 0)),
            pl.BlockSpec((1, C), lambda i: (0, 0)),
            pl.BlockSpec((1, HID), lambda i: (0, 0)),
            pl.BlockSpec((1, HID), lambda i: (0, 0)),
            pl.BlockSpec((HID, HID), lambda i: (0, 0)),
            pl.BlockSpec((1, HID), lambda i: (0, 0)),
        ],
        out_specs=pl.BlockSpec((_QB, C), lambda i: (i, 0)),
        out_shape=jax.ShapeDtypeStruct((N, C), jnp.float32),
    )(pg3, p, w1, xvg3, Wp1T, bp1, s1, t1, Wp2T, bp2, s3, t3, Ww2T, bw2)


def _bn_affine(sums, gamma, beta, count):
    s1, s2 = sums
    mean = s1 / count
    var = s2 / count - mean * mean
    scale = gamma * lax.rsqrt(var + EPS)
    shift = beta - mean * scale
    return scale, shift


def kernel(p, x, o, params):
    del o
    WqT = params["Wq"].T
    WkT = params["Wk"].T
    WvT = params["Wv"].T
    bq = params["bq"].reshape(1, C)
    bk = params["bk"].reshape(1, C)
    bv = params["bv"].reshape(1, C)
    Wp1T = params["Wp1"].T
    bp1 = params["bp1"].reshape(1, 3)
    Wp2T = params["Wp2"].T
    bp2 = params["bp2"].reshape(1, C)
    Ww1T = params["Ww1"].T
    bw1 = params["bw1"].reshape(1, HID)
    Ww2T = params["Ww2"].T
    bw2 = params["bw2"].reshape(1, HID)

    xq, xk, xv = _qkv(x, WqT, bq, WkT, bk, WvT, bv)
    idx = _knn(p, p.T)

    p128 = jnp.concatenate([p, jnp.zeros((N, 125), jnp.float32)], axis=1)
    xkg, xvg, pgw = _sc_gather(xk, xv, p128, idx.reshape(EDGES))
    pg = pgw[:, :16]
    pg3 = pg.reshape(N, NS, 16)
    xkg3 = xkg.reshape(N, NS, C)
    xvg3 = xvg.reshape(N, NS, C)

    cnt = float(EDGES)
    st1 = _hstats(pg3, p, Wp1T, bp1)
    sc1, sh1 = _bn_affine((st1[0, 0:3], st1[1, 0:3]), params["gp"], params["bnp"], cnt)
    s1 = sc1.reshape(1, 3)
    t1 = sh1.reshape(1, 3)

    st2 = _rstats(pg3, p, xq, xkg3, Wp1T, bp1, s1, t1, Wp2T, bp2)
    sc2, sh2 = _bn_affine((st2[0, :], st2[1, :]), params["gw1"], params["bnw1"], cnt)
    s2 = sc2.reshape(1, C)
    t2 = sh2.reshape(1, C)

    w1, st3 = _w1pass(pg3, p, xq, xkg3, Wp1T, bp1, s1, t1, Wp2T, bp2, s2, t2, Ww1T, bw1)
    sc3, sh3 = _bn_affine((st3[0, 0:HID], st3[1, 0:HID]), params["gw2"], params["bnw2"], cnt)
    s3 = sc3.reshape(1, HID)
    t3 = sh3.reshape(1, HID)

    out = _final(pg3, p, w1, xvg3, Wp1T, bp1, s1, t1, Wp2T, bp2, s3, t3, Ww2T, bw2)
    return out


# speedup vs baseline: 10.1164x; 1.1035x over previous
import functools

import jax
import jax.numpy as jnp
from jax import lax
from jax.experimental import pallas as pl
from jax.experimental.pallas import tpu as pltpu
from jax.experimental.pallas import tpu_sc as plsc

N = 8192
C = 256
NS = 16
SEG = 2048
NSEG = 4
HID = 32
SHARE = 8
EPS = 1e-5
EDGES = N * NS

_QKV_BLK = 256
_KNN_QB = 256
_QB = 128


def _qkv_body(x_ref, wq_ref, bq_ref, wk_ref, bk_ref, wv_ref, bv_ref,
              xq_ref, xk_ref, xv_ref):
    x = x_ref[...]
    xq_ref[...] = jnp.dot(x, wq_ref[...], preferred_element_type=jnp.float32) + bq_ref[...]
    xk_ref[...] = jnp.dot(x, wk_ref[...], preferred_element_type=jnp.float32) + bk_ref[...]
    xv_ref[...] = jnp.dot(x, wv_ref[...], preferred_element_type=jnp.float32) + bv_ref[...]


def _qkv(x, WqT, bq, WkT, bk, WvT, bv):
    grid = N // _QKV_BLK
    blk = pl.BlockSpec((_QKV_BLK, C), lambda i: (i, 0))
    wspec = pl.BlockSpec((C, C), lambda i: (0, 0))
    bspec = pl.BlockSpec((1, C), lambda i: (0, 0))
    return pl.pallas_call(
        _qkv_body,
        grid=(grid,),
        in_specs=[blk, wspec, bspec, wspec, bspec, wspec, bspec],
        out_specs=[blk, blk, blk],
        out_shape=[jax.ShapeDtypeStruct((N, C), jnp.float32)] * 3,
    )(x, WqT, bq, WkT, bk, WvT, bv)


def _knn_body(p_ref, pT_ref, idx_ref):
    s = pl.program_id(0)
    q = p_ref[...]
    sT = pT_ref[...]
    sq_q = q[:, 0:1] * q[:, 0:1] + q[:, 1:2] * q[:, 1:2] + q[:, 2:3] * q[:, 2:3]
    sq_s = (sT[0:1, :] * sT[0:1, :] + sT[1:2, :] * sT[1:2, :]
            + sT[2:3, :] * sT[2:3, :])
    dot = jnp.dot(q.astype(jnp.bfloat16), sT.astype(jnp.bfloat16),
                  preferred_element_type=jnp.float32)
    d2 = sq_q + sq_s - 2.0 * dot
    loc = lax.broadcasted_iota(jnp.int32, (_KNN_QB, SEG), 1)
    big = jnp.int32(2 ** 30)
    cols = []
    for _ in range(NS):
        m = jnp.min(d2, axis=1, keepdims=True)
        idxc = jnp.min(jnp.where(d2 == m, loc, big), axis=1, keepdims=True)
        cols.append(idxc + s * SEG)
        d2 = jnp.where(loc == idxc, jnp.inf, d2)
    idx_ref[...] = jnp.concatenate(cols, axis=1)


def _knn(p, pT):
    qpb = SEG // _KNN_QB
    return pl.pallas_call(
        _knn_body,
        grid=(NSEG, qpb),
        in_specs=[
            pl.BlockSpec((_KNN_QB, 3), lambda s, j: (s * qpb + j, 0)),
            pl.BlockSpec((3, SEG), lambda s, j: (0, s)),
        ],
        out_specs=pl.BlockSpec((_KNN_QB, NS), lambda s, j: (s * qpb + j, 0)),
        out_shape=jax.ShapeDtypeStruct((N, NS), jnp.int32),
    )(p, pT)


_GCHUNK = 128
_NW = 32


def _sc_gather_kp(xk, p128, idxf):
    per_w = EDGES // _NW
    ch = 64
    nch = per_w // ch
    mesh = plsc.VectorSubcoreMesh(core_axis_name="c", subcore_axis_name="s")

    @functools.partial(
        pl.kernel,
        mesh=mesh,
        out_type=[
            jax.ShapeDtypeStruct((EDGES, C), jnp.float32),
            jax.ShapeDtypeStruct((EDGES, 16), jnp.float32),
        ],
        scratch_types=[
            pltpu.VMEM((2, ch), jnp.int32),
            pltpu.VMEM((2, ch, C), jnp.float32),
            pltpu.VMEM((2, ch, 128), jnp.float32),
            pltpu.VMEM((2, ch, 16), jnp.float32),
            pltpu.SemaphoreType.DMA,
            pltpu.SemaphoreType.DMA,
            pltpu.SemaphoreType.DMA,
            pltpu.SemaphoreType.DMA,
        ],
    )
    def gk(xk_hbm, p_hbm, idx_hbm, xkg_hbm, pg_hbm,
           idx_v, bufk, bufp, bufpn, g0, g1, w0, w1):
        wid = lax.axis_index("s") * 2 + lax.axis_index("c")
        gsem = (g0, g1)
        wsem = (w0, w1)

        def start_g(c, b):
            pltpu.sync_copy(idx_hbm.at[pl.ds(wid * per_w + c * ch, ch)],
                            idx_v.at[b])
            pltpu.async_copy(xk_hbm.at[idx_v.at[b]], bufk.at[b], gsem[b])
            pltpu.async_copy(p_hbm.at[idx_v.at[b]], bufp.at[b], gsem[b])

        def wait_g(b):
            pltpu.make_async_copy(xk_hbm.at[idx_v.at[b]], bufk.at[b], gsem[b]).wait()
            pltpu.make_async_copy(p_hbm.at[idx_v.at[b]], bufp.at[b], gsem[b]).wait()

        def compact(b):
            def cbody(r, carry):
                bufpn[b, r, :] = bufp[b, r, 0:16]
                return carry
            lax.fori_loop(0, ch, cbody, 0)

        def start_w(c, b):
            base = wid * per_w + c * ch
            pltpu.async_copy(bufk.at[b], xkg_hbm.at[pl.ds(base, ch)], wsem[b])
            pltpu.async_copy(bufpn.at[b], pg_hbm.at[pl.ds(base, ch)], wsem[b])

        def wait_w(c, b):
            base = wid * per_w + c * ch
            pltpu.make_async_copy(bufk.at[b], xkg_hbm.at[pl.ds(base, ch)], wsem[b]).wait()
            pltpu.make_async_copy(bufpn.at[b], pg_hbm.at[pl.ds(base, ch)], wsem[b]).wait()

        def body(j, carry):
            for b in (0, 1):
                c = 2 * j + b

                @pl.when(c >= 2)
                def _():
                    wait_w(c - 2, b)

                start_g(c, b)

                @pl.when(c >= 1)
                def _():
                    wait_g(1 - b)
                    compact(1 - b)
                    start_w(c - 1, 1 - b)
            return carry

        lax.fori_loop(0, nch // 2, body, 0)
        wait_g(1)
        compact(1)
        start_w(nch - 1, 1)
        wait_w(nch - 2, 0)
        wait_w(nch - 1, 1)

    return gk(xk, p128, idxf)


def _sc_gather_v(xv, idxf):
    per_w = EDGES // _NW
    nch = per_w // _GCHUNK
    mesh = plsc.VectorSubcoreMesh(core_axis_name="c", subcore_axis_name="s")

    @functools.partial(
        pl.kernel,
        mesh=mesh,
        out_type=jax.ShapeDtypeStruct((EDGES, C), jnp.float32),
        scratch_types=[
            pltpu.VMEM((2, _GCHUNK), jnp.int32),
            pltpu.VMEM((2, _GCHUNK, C), jnp.float32),
            pltpu.SemaphoreType.DMA,
            pltpu.SemaphoreType.DMA,
            pltpu.SemaphoreType.DMA,
            pltpu.SemaphoreType.DMA,
        ],
    )
    def gk(xv_hbm, idx_hbm, xvg_hbm, idx_v, bufv, g0, g1, w0, w1):
        wid = lax.axis_index("s") * 2 + lax.axis_index("c")
        gsem = (g0, g1)
        wsem = (w0, w1)

        def start_g(c, b):
            pltpu.sync_copy(idx_hbm.at[pl.ds(wid * per_w + c * _GCHUNK, _GCHUNK)],
                            idx_v.at[b])
            pltpu.async_copy(xv_hbm.at[idx_v.at[b]], bufv.at[b], gsem[b])

        def wait_g(b):
            pltpu.make_async_copy(xv_hbm.at[idx_v.at[b]], bufv.at[b], gsem[b]).wait()

        def start_w(c, b):
            base = wid * per_w + c * _GCHUNK
            pltpu.async_copy(bufv.at[b], xvg_hbm.at[pl.ds(base, _GCHUNK)], wsem[b])

        def wait_w(c, b):
            base = wid * per_w + c * _GCHUNK
            pltpu.make_async_copy(bufv.at[b], xvg_hbm.at[pl.ds(base, _GCHUNK)], wsem[b]).wait()

        def body(j, carry):
            for b in (0, 1):
                c = 2 * j + b

                @pl.when(c >= 2)
                def _():
                    wait_w(c - 2, b)

                start_g(c, b)

                @pl.when(c >= 1)
                def _():
                    wait_g(1 - b)
                    start_w(c - 1, 1 - b)
            return carry

        lax.fori_loop(0, nch // 2, body, 0)
        wait_g(1)
        start_w(nch - 1, 1)
        wait_w(nch - 2, 0)
        wait_w(nch - 1, 1)

    return gk(xv, idxf)


def _hstats_body(pg_ref, p_ref, wp1t_ref, bp1_ref, st_ref):
    @pl.when(pl.program_id(0) == 0)
    def _():
        st_ref[...] = jnp.zeros_like(st_ref)

    qb = p_ref.shape[0]
    p_r = pg_ref[:, :, :3] - jnp.reshape(p_ref[...], (qb, 1, 3))
    h = jnp.dot(p_r.reshape(qb * NS, 3), wp1t_ref[...],
                preferred_element_type=jnp.float32) + bp1_ref[...]
    st_ref[0:1, 0:3] += jnp.sum(h, axis=0, keepdims=True)
    st_ref[1:2, 0:3] += jnp.sum(h * h, axis=0, keepdims=True)


def _hstats(pg3, p, Wp1T, bp1):
    grid = N // 256
    return pl.pallas_call(
        _hstats_body,
        grid=(grid,),
        in_specs=[
            pl.BlockSpec((256, NS, 16), lambda i: (i, 0, 0)),
            pl.BlockSpec((256, 3), lambda i: (i, 0)),
            pl.BlockSpec((3, 3), lambda i: (0, 0)),
            pl.BlockSpec((1, 3), lambda i: (0, 0)),
        ],
        out_specs=pl.BlockSpec((8, 128), lambda i: (0, 0)),
        out_shape=jax.ShapeDtypeStruct((8, 128), jnp.float32),
    )(pg3, p, Wp1T, bp1)


def _pe_block(pg_b, p_b, wp1t, bp1, s1, t1, wp2t, bp2):
    qb = p_b.shape[0]
    p_r = pg_b[:, :, :3] - jnp.reshape(p_b, (qb, 1, 3))
    h = jnp.dot(p_r.reshape(qb * NS, 3), wp1t,
                preferred_element_type=jnp.float32) + bp1
    h = jnp.maximum(h * s1 + t1, 0.0)
    pe = jnp.dot(h, wp2t, preferred_element_type=jnp.float32) + bp2
    return pe.reshape(qb, NS, C)


def _rstats_body(pg_ref, p_ref, xq_ref, xkg_ref, wp1t_ref, bp1_ref,
                 s1_ref, t1_ref, wp2t_ref, bp2_ref, st_ref):
    @pl.when(pl.program_id(0) == 0)
    def _():
        st_ref[...] = jnp.zeros_like(st_ref)

    pe = _pe_block(pg_ref[...], p_ref[...], wp1t_ref[...], bp1_ref[...],
                   s1_ref[...], t1_ref[...], wp2t_ref[...], bp2_ref[...])
    r = xkg_ref[...] - jnp.reshape(xq_ref[...], (_QB, 1, C)) + pe
    r2 = r.reshape(_QB * NS, C)
    st_ref[0:1, :] += jnp.sum(r2, axis=0, keepdims=True)
    st_ref[1:2, :] += jnp.sum(r2 * r2, axis=0, keepdims=True)


def _rstats(pg3, p, xq, xkg3, Wp1T, bp1, s1, t1, Wp2T, bp2):
    grid = N // _QB
    return pl.pallas_call(
        _rstats_body,
        grid=(grid,),
        in_specs=[
            pl.BlockSpec((_QB, NS, 16), lambda i: (i, 0, 0)),
            pl.BlockSpec((_QB, 3), lambda i: (i, 0)),
            pl.BlockSpec((_QB, C), lambda i: (i, 0)),
            pl.BlockSpec((_QB, NS, C), lambda i: (i, 0, 0)),
            pl.BlockSpec((3, 3), lambda i: (0, 0)),
            pl.BlockSpec((1, 3), lambda i: (0, 0)),
            pl.BlockSpec((1, 3), lambda i: (0, 0)),
            pl.BlockSpec((1, 3), lambda i: (0, 0)),
            pl.BlockSpec((3, C), lambda i: (0, 0)),
            pl.BlockSpec((1, C), lambda i: (0, 0)),
        ],
        out_specs=pl.BlockSpec((8, C), lambda i: (0, 0)),
        out_shape=jax.ShapeDtypeStruct((8, C), jnp.float32),
    )(pg3, p, xq, xkg3, Wp1T, bp1, s1, t1, Wp2T, bp2)


def _w1_body(pg_ref, p_ref, xq_ref, xkg_ref, wp1t_ref, bp1_ref,
             s1_ref, t1_ref, wp2t_ref, bp2_ref, s2_ref, t2_ref,
             ww1t_ref, bw1_ref, w1_ref, st_ref):
    @pl.when(pl.program_id(0) == 0)
    def _():
        st_ref[...] = jnp.zeros_like(st_ref)

    pe = _pe_block(pg_ref[...], p_ref[...], wp1t_ref[...], bp1_ref[...],
                   s1_ref[...], t1_ref[...], wp2t_ref[...], bp2_ref[...])
    r = xkg_ref[...] - jnp.reshape(xq_ref[...], (_QB, 1, C)) + pe
    r2 = r.reshape(_QB * NS, C)
    w = jnp.maximum(r2 * s2_ref[...] + t2_ref[...], 0.0)
    w1 = jnp.dot(w, ww1t_ref[...], preferred_element_type=jnp.float32) + bw1_ref[...]
    w1_ref[...] = w1.reshape(_QB, NS, HID)
    st_ref[0:1, 0:HID] += jnp.sum(w1, axis=0, keepdims=True)
    st_ref[1:2, 0:HID] += jnp.sum(w1 * w1, axis=0, keepdims=True)


def _w1pass(pg3, p, xq, xkg3, Wp1T, bp1, s1, t1, Wp2T, bp2, s2, t2, Ww1T, bw1):
    grid = N // _QB
    return pl.pallas_call(
        _w1_body,
        grid=(grid,),
        in_specs=[
            pl.BlockSpec((_QB, NS, 16), lambda i: (i, 0, 0)),
            pl.BlockSpec((_QB, 3), lambda i: (i, 0)),
            pl.BlockSpec((_QB, C), lambda i: (i, 0)),
            pl.BlockSpec((_QB, NS, C), lambda i: (i, 0, 0)),
            pl.BlockSpec((3, 3), lambda i: (0, 0)),
            pl.BlockSpec((1, 3), lambda i: (0, 0)),
            pl.BlockSpec((1, 3), lambda i: (0, 0)),
            pl.BlockSpec((1, 3), lambda i: (0, 0)),
            pl.BlockSpec((3, C), lambda i: (0, 0)),
            pl.BlockSpec((1, C), lambda i: (0, 0)),
            pl.BlockSpec((1, C), lambda i: (0, 0)),
            pl.BlockSpec((1, C), lambda i: (0, 0)),
            pl.BlockSpec((C, HID), lambda i: (0, 0)),
            pl.BlockSpec((1, HID), lambda i: (0, 0)),
        ],
        out_specs=[
            pl.BlockSpec((_QB, NS, HID), lambda i: (i, 0, 0)),
            pl.BlockSpec((8, 128), lambda i: (0, 0)),
        ],
        out_shape=[
            jax.ShapeDtypeStruct((N, NS, HID), jnp.float32),
            jax.ShapeDtypeStruct((8, 128), jnp.float32),
        ],
    )(pg3, p, xq, xkg3, Wp1T, bp1, s1, t1, Wp2T, bp2, s2, t2, Ww1T, bw1)


def _final_body(pg_ref, p_ref, w1_ref, xvg_ref, wp1t_ref, bp1_ref,
                s1_ref, t1_ref, wp2t_ref, bp2_ref, s3_ref, t3_ref,
                ww2t_ref, bw2_ref, out_ref):
    y = jnp.maximum(w1_ref[...].reshape(_QB * NS, HID) * s3_ref[...] + t3_ref[...], 0.0)
    w2 = jnp.dot(y, ww2t_ref[...], preferred_element_type=jnp.float32) + bw2_ref[...]
    w2 = w2.reshape(_QB, NS, HID)
    m = jnp.max(w2, axis=1, keepdims=True)
    e = jnp.exp(w2 - m)
    sm = e / jnp.sum(e, axis=1, keepdims=True)
    pe = _pe_block(pg_ref[...], p_ref[...], wp1t_ref[...], bp1_ref[...],
                   s1_ref[...], t1_ref[...], wp2t_ref[...], bp2_ref[...])
    xvp = xvg_ref[...] + pe
    wf = jnp.concatenate([sm] * SHARE, axis=2)
    out_ref[...] = jnp.sum(xvp * wf, axis=1)


def _final(pg3, p, w1, xvg3, Wp1T, bp1, s1, t1, Wp2T, bp2, s3, t3, Ww2T, bw2):
    grid = N // _QB
    return pl.pallas_call(
        _final_body,
        grid=(grid,),
        in_specs=[
            pl.BlockSpec((_QB, NS, 16), lambda i: (i, 0, 0)),
            pl.BlockSpec((_QB, 3), lambda i: (i, 0)),
            pl.BlockSpec((_QB, NS, HID), lambda i: (i, 0, 0)),
            pl.BlockSpec((_QB, NS, C), lambda i: (i, 0, 0)),
            pl.BlockSpec((3, 3), lambda i: (0, 0)),
            pl.BlockSpec((1, 3), lambda i: (0, 0)),
            pl.BlockSpec((1, 3), lambda i: (0, 0)),
            pl.BlockSpec((1, 3), lambda i: (0, 0)),
            pl.BlockSpec((3, C), lambda i: (0, 0)),
            pl.BlockSpec((1, C), lambda i: (0, 0)),
            pl.BlockSpec((1, HID), lambda i: (0, 0)),
            pl.BlockSpec((1, HID), lambda i: (0, 0)),
            pl.BlockSpec((HID, HID), lambda i: (0, 0)),
            pl.BlockSpec((1, HID), lambda i: (0, 0)),
        ],
        out_specs=pl.BlockSpec((_QB, C), lambda i: (i, 0)),
        out_shape=jax.ShapeDtypeStruct((N, C), jnp.float32),
    )(pg3, p, w1, xvg3, Wp1T, bp1, s1, t1, Wp2T, bp2, s3, t3, Ww2T, bw2)


def _bn_affine(sums, gamma, beta, count):
    s1, s2 = sums
    mean = s1 / count
    var = s2 / count - mean * mean
    scale = gamma * lax.rsqrt(var + EPS)
    shift = beta - mean * scale
    return scale, shift


def kernel(p, x, o, params):
    del o
    WqT = params["Wq"].T
    WkT = params["Wk"].T
    WvT = params["Wv"].T
    bq = params["bq"].reshape(1, C)
    bk = params["bk"].reshape(1, C)
    bv = params["bv"].reshape(1, C)
    Wp1T = params["Wp1"].T
    bp1 = params["bp1"].reshape(1, 3)
    Wp2T = params["Wp2"].T
    bp2 = params["bp2"].reshape(1, C)
    Ww1T = params["Ww1"].T
    bw1 = params["bw1"].reshape(1, HID)
    Ww2T = params["Ww2"].T
    bw2 = params["bw2"].reshape(1, HID)

    xq, xk, xv = _qkv(x, WqT, bq, WkT, bk, WvT, bv)
    idx = _knn(p, p.T)

    p128 = jnp.concatenate([p, jnp.zeros((N, 125), jnp.float32)], axis=1)
    idxf = idx.reshape(EDGES)
    xkg, pg = _sc_gather_kp(xk, p128, idxf)
    xvg = _sc_gather_v(xv, idxf)
    pg3 = pg.reshape(N, NS, 16)
    xkg3 = xkg.reshape(N, NS, C)
    xvg3 = xvg.reshape(N, NS, C)

    cnt = float(EDGES)
    st1 = _hstats(pg3, p, Wp1T, bp1)
    sc1, sh1 = _bn_affine((st1[0, 0:3], st1[1, 0:3]), params["gp"], params["bnp"], cnt)
    s1 = sc1.reshape(1, 3)
    t1 = sh1.reshape(1, 3)

    st2 = _rstats(pg3, p, xq, xkg3, Wp1T, bp1, s1, t1, Wp2T, bp2)
    sc2, sh2 = _bn_affine((st2[0, :], st2[1, :]), params["gw1"], params["bnw1"], cnt)
    s2 = sc2.reshape(1, C)
    t2 = sh2.reshape(1, C)

    w1, st3 = _w1pass(pg3, p, xq, xkg3, Wp1T, bp1, s1, t1, Wp2T, bp2, s2, t2, Ww1T, bw1)
    sc3, sh3 = _bn_affine((st3[0, 0:HID], st3[1, 0:HID]), params["gw2"], params["bnw2"], cnt)
    s3 = sc3.reshape(1, HID)
    t3 = sh3.reshape(1, HID)

    out = _final(pg3, p, w1, xvg3, Wp1T, bp1, s1, t1, Wp2T, bp2, s3, t3, Ww2T, bw2)
    return out
